# baseline (speedup 1.0000x reference)
import functools

import jax
import jax.numpy as jnp
import numpy as np
from jax import lax
from jax.experimental import pallas as pl
from jax.experimental.pallas import tpu as pltpu
from jax.experimental.pallas import tpu_sc as plsc

N1, N2, NQ, NL = 2000, 2000, 4, 3
NP_, NN_, NZ_ = 100000, 100000, 800000
LOG2PI = float(np.log(2.0 * np.pi))

NC, NS = 2, 16
NW = NC * NS
SZ = 25008
SPN = 3136
BZ = SZ // 16
BPN = SPN // 16

_LOGC = (-7.788711203771115e-10, 0.9999999096875468, -0.49999956288490294,
         0.3333465033023849, -0.2500434801612538, 0.19950129931010036,
         -0.16511500339293791, 0.14932157138841426, -0.14707770124455374,
         0.09274272935687101)
_LN2 = 0.6931471805599453
_OFF = 0x3F330000


def _plog(g):
    bits = lax.bitcast_convert_type(g, jnp.int32)
    tmp = bits - jnp.int32(_OFF)
    k = tmp >> 23
    z = lax.bitcast_convert_type(bits - (tmp & jnp.int32(-8388608)), jnp.float32)
    u = z - 1.0
    acc = jnp.full(g.shape, _LOGC[-1], jnp.float32)
    for i in range(len(_LOGC) - 2, -1, -1):
        acc = acc * u + jnp.float32(_LOGC[i])
    return k.astype(jnp.float32) * jnp.float32(_LN2) + acc


def _tc_prep(g8_ref, t1_ref, t2_ref, sc_ref, tab_ref, dense_ref):
    g8 = g8_ref[...]
    ga, gra, gb, grb = g8[0:1], g8[1:2], g8[2:3], g8[3:4]
    gp, grp, gq, grq = g8[4:5], g8[5:6], g8[6:7], g8[7:8]
    rho_a, rho_b = jnp.exp(gra), jnp.exp(grb)
    rho_p, rho_q = jnp.exp(grp), jnp.exp(grq)

    zrow = jnp.zeros((1, N1), jnp.float32)
    t1 = jnp.concatenate([t1_ref[...], zrow], axis=0)
    mx1 = jnp.max(t1, axis=0, keepdims=True)
    e1 = jnp.exp(t1 - mx1)
    z1 = jnp.sum(e1, axis=0, keepdims=True)
    tau1 = e1 / z1
    S1 = jnp.sum(tau1 * (t1 - mx1 - jnp.log(z1)))

    t2 = jnp.concatenate([t2_ref[...], zrow], axis=0)
    mx2 = jnp.max(t2, axis=0, keepdims=True)
    e2 = jnp.exp(t2 - mx2)
    z2 = jnp.sum(e2, axis=0, keepdims=True)
    tau2 = e2 / z2
    S2 = jnp.sum(tau2 * (t2 - mx2 - jnp.log(z2)))

    entropy = (0.5 * (2 * N1 + 2 * N2) * (LOG2PI + 1.0)
               + 0.5 * (jnp.sum(gra) + jnp.sum(grb) + jnp.sum(grp) + jnp.sum(grq))
               - S1 - S2)
    ell_a = -N1 / 2 * (LOG2PI + sc_ref[1]) - sc_ref[5] * jnp.sum(rho_a + ga * ga)
    ell_b = -N1 / 2 * (LOG2PI + sc_ref[2]) - sc_ref[6] * jnp.sum(rho_b + gb * gb)
    ell_p = -N2 / 2 * (LOG2PI + sc_ref[3]) - sc_ref[7] * jnp.sum(rho_p + gp * gp)
    ell_q = -N2 / 2 * (LOG2PI + sc_ref[4]) - sc_ref[8] * jnp.sum(rho_q + gq * gq)
    ell_y1 = sum(jnp.sum(tau1[q:q + 1, :]) * sc_ref[33 + q] for q in range(NQ))
    ell_y2 = sum(jnp.sum(tau2[l:l + 1, :]) * sc_ref[37 + l] for l in range(NL))

    Ap = [sum(tau1[q:q + 1, :] * sc_ref[9 + q * NL + l] for q in range(NQ))
          for l in range(NL)]
    Am = [sum(tau1[q:q + 1, :] * sc_ref[21 + q * NL + l] for q in range(NQ))
          for l in range(NL)]

    tab_ref[...] = jnp.concatenate(
        [ga, gb, rho_a, rho_b, tau1, gp, gq, rho_p, rho_q, tau2] + Ap + Am,
        axis=0)
    dense_ref[0] = entropy + ell_a + ell_b + ell_p + ell_q + ell_y1 + ell_y2


def _sc_body(tab_h, c16_h, ip_h, jp_h, in_h, jn_h, iz_h, jz_h, out_h,
             tab_v, c_v, iz_v, jz_v, ip_v, jp_v, in_v, jn_v, acc_v):
    wid = lax.axis_index("s") * NC + lax.axis_index("c")
    pltpu.sync_copy(tab_h, tab_v)
    pltpu.sync_copy(c16_h, c_v)
    pltpu.sync_copy(iz_h.at[pl.ds(wid * SZ, SZ)], iz_v)
    pltpu.sync_copy(jz_h.at[pl.ds(wid * SZ, SZ)], jz_v)
    pltpu.sync_copy(ip_h.at[pl.ds(wid * SPN, SPN)], ip_v)
    pltpu.sync_copy(jp_h.at[pl.ds(wid * SPN, SPN)], jp_v)
    pltpu.sync_copy(in_h.at[pl.ds(wid * SPN, SPN)], in_v)
    pltpu.sync_copy(jn_h.at[pl.ds(wid * SPN, SPN)], jn_v)

    def full(v):
        return jnp.full((16,), v, jnp.int32)

    def gath(row, ii):
        return plsc.load_gather(tab_v, [full(row), ii])


    mu_v = c_v[pl.ds(0, 16)]
    pi_vs = [c_v[pl.ds((1 + c) * 16, 16)] for c in range(NQ * NL)]
    iota = lax.iota(jnp.int32, 16)

    validz = NZ_ - wid * SZ

    def zbody(k, acc):
        ii = iz_v[pl.ds(k * 16, 16)]
        jj = jz_v[pl.ds(k * 16, 16)]
        na, nb = gath(0, ii), gath(1, ii)
        ra, rb = gath(2, ii), gath(3, ii)
        t1 = [gath(4 + q, ii) for q in range(NQ)]
        np_, nq_ = gath(8, jj), gath(9, jj)
        rp, rq = gath(10, jj), gath(11, jj)
        t2 = [gath(12 + l, jj) for l in range(NL)]
        x = na + np_
        y = nb + nq_
        svx = ra + rp
        svy = rb + rq
        s1 = 1.0 / (1.0 + jnp.exp(-(mu_v + x + y)))
        s2 = 1.0 / (1.0 + jnp.exp(-(mu_v + x - y)))
        u1 = s1 * (1.0 - s1)
        w1 = u1 * (1.0 - 2.0 * s1)
        u2 = s2 * (1.0 - s2)
        w2 = u2 * (1.0 - 2.0 * s2)
        A = 1.0 - s2
        d = s1 - s2
        nw2 = -w2
        ndw = w2 - w1
        nu2 = -u2
        ndu = u2 - u1
        nsu = -(u1 + u2)
        svs = 0.5 * (svx + svy)
        hx = 0.5 * svx
        hy = 0.5 * svy
        big = jnp.abs(d) >= 1e-5
        invd = 1.0 / jnp.where(big, d, 1.0)
        ndw_e = jnp.where(big, ndw, 0.0)
        ndu_e = jnp.where(big, ndu, 0.0)
        nsu_e = jnp.where(big, nsu, 0.0)
        Ad = A * invd
        C1 = -(ndw_e * invd)
        C2 = nw2 + ndw_e * Ad
        E1 = -(ndu_e * invd)
        E2 = nu2 + ndu_e * Ad
        F1 = -(nsu_e * invd)
        F2 = u2 + nsu_e * Ad
        hx2 = hx + hx
        hy2 = hy + hy
        P0 = svs * C1 - hx * (E1 * E1) - hy * (F1 * F1)
        P1 = svs * C2 - hx2 * (E1 * E2) - hy2 * (F1 * F2)
        P2 = -(hx * (E2 * E2) + hy * (F2 * F2))
        blk = jnp.zeros((16,), jnp.float32)
        for c in range(NQ * NL):
            piv = pi_vs[c]
            g = A - piv * d
            lg = _plog(g)
            r = 1.0 / g
            blk = blk + (t1[c // NL] * t2[c % NL]) * ((lg + P0) + (P1 + P2 * r) * r)
        return acc + jnp.where(iota < validz - k * 16, blk, 0.0)

    acc = plsc.parallel_loop(0, BZ, unroll=8,
                             carry=jnp.zeros((16,), jnp.float32))(
        lambda k, a: zbody(k, a))

    def make_pn(iv, jv, valid, arow, neg):
        def body(k, acc):
            ii = iv[pl.ds(k * 16, 16)]
            jj = jv[pl.ds(k * 16, 16)]
            na, nb = gath(0, ii), gath(1, ii)
            ra, rb = gath(2, ii), gath(3, ii)
            np_, nq_ = gath(8, jj), gath(9, jj)
            rp, rq = gath(10, jj), gath(11, jj)
            t2 = [gath(12 + l, jj) for l in range(NL)]
            A = [gath(arow + l, ii) for l in range(NL)]
            if neg:
                z = mu_v + na + np_ - nb - nq_
            else:
                z = mu_v + na + np_ + nb + nq_
            ez = jnp.exp(-jnp.abs(z))
            inv = 1.0 / (1.0 + ez)
            s = jnp.where(z >= 0, inv, ez * inv)
            lsg = jnp.minimum(z, 0.0) - _plog(1.0 + ez)
            der2 = -s * (1.0 - s)
            sv = ra + rb + rp + rq
            dotA = A[0] * t2[0] + A[1] * t2[1] + A[2] * t2[2]
            val = dotA + lsg + 0.5 * der2 * sv
            return acc + jnp.where(iota < valid - k * 16, val, 0.0)
        return body

    acc = plsc.parallel_loop(0, BPN, unroll=4, carry=acc)(
        make_pn(ip_v, jp_v, NP_ - wid * SPN, 15, False))
    acc = plsc.parallel_loop(0, BPN, unroll=4, carry=acc)(
        make_pn(in_v, jn_v, NN_ - wid * SPN, 18, True))

    acc_v[...] = acc
    pltpu.sync_copy(acc_v, out_h.at[pl.ds(wid * 16, 16)])


def _pad_idx(a, total):
    a = a.astype(jnp.int32)
    return jnp.pad(a, (0, total - a.shape[0]))


@jax.jit
def kernel(variationnal_params, model_params, i_p, j_p, i_n, j_n, i_z, j_z):
    vp = variationnal_params.astype(jnp.float32)
    mp = model_params.astype(jnp.float32)

    g8 = jnp.stack([vp[0:N1], vp[N1:2 * N1], vp[2 * N1:3 * N1], vp[3 * N1:4 * N1],
                    vp[4 * N1:4 * N1 + N2], vp[4 * N1 + N2:4 * N1 + 2 * N2],
                    vp[4 * N1 + 2 * N2:4 * N1 + 3 * N2],
                    vp[4 * N1 + 3 * N2:4 * N1 + 4 * N2]])
    o = 4 * N1 + 4 * N2
    t1T = vp[o:o + N1 * (NQ - 1)].reshape(N1, NQ - 1).T
    o += N1 * (NQ - 1)
    t2T = vp[o:o + N2 * (NL - 1)].reshape(N2, NL - 1).T

    mu = mp[0]
    inv2 = 0.5 * jnp.exp(-mp[1:5])
    la1 = jax.nn.log_softmax(jnp.pad(mp[5:5 + NQ - 1], (0, 1)))
    la2 = jax.nn.log_softmax(jnp.pad(mp[5 + NQ - 1:5 + NQ + NL - 2], (0, 1)))
    pi = jax.nn.sigmoid(mp[10:10 + NQ * NL])
    sc = jnp.concatenate([mp[0:5], inv2, jnp.log(pi), jnp.log1p(-pi), la1, la2])
    c16 = jnp.repeat(jnp.concatenate([mu[None], pi]), 16)

    tab, dense = pl.pallas_call(
        _tc_prep,
        in_specs=[pl.BlockSpec(memory_space=pltpu.VMEM),
                  pl.BlockSpec(memory_space=pltpu.VMEM),
                  pl.BlockSpec(memory_space=pltpu.VMEM),
                  pl.BlockSpec(memory_space=pltpu.SMEM)],
        out_specs=[pl.BlockSpec(memory_space=pltpu.VMEM),
                   pl.BlockSpec(memory_space=pltpu.SMEM)],
        out_shape=[jax.ShapeDtypeStruct((21, N1), jnp.float32),
                   jax.ShapeDtypeStruct((1,), jnp.float32)],
    )(g8, t1T, t2T, sc)

    sc_fn = pl.kernel(
        _sc_body,
        out_type=jax.ShapeDtypeStruct((NW * 16,), jnp.float32),
        mesh=plsc.VectorSubcoreMesh(core_axis_name="c", subcore_axis_name="s",
                                    num_cores=NC, num_subcores=NS),
        compiler_params=pltpu.CompilerParams(use_tc_tiling_on_sc=False, needs_layout_passes=False),
        scratch_types=[
            pltpu.VMEM((21, N1), jnp.float32),
            pltpu.VMEM((13 * 16,), jnp.float32),
            pltpu.VMEM((SZ,), jnp.int32),
            pltpu.VMEM((SZ,), jnp.int32),
            pltpu.VMEM((SPN,), jnp.int32),
            pltpu.VMEM((SPN,), jnp.int32),
            pltpu.VMEM((SPN,), jnp.int32),
            pltpu.VMEM((SPN,), jnp.int32),
            pltpu.VMEM((16,), jnp.float32),
        ],
    )
    partials = sc_fn(tab, c16,
                     _pad_idx(i_p, NW * SPN), _pad_idx(j_p, NW * SPN),
                     _pad_idx(i_n, NW * SPN), _pad_idx(j_n, NW * SPN),
                     _pad_idx(i_z, NW * SZ), _pad_idx(j_z, NW * SZ))

    exp_x = jnp.sum(partials)
    exp_x = jnp.where(exp_x < 0, exp_x, jnp.inf)
    return -(dense + exp_x)


# speedup vs baseline: 1.1290x; 1.1290x over previous
import functools

import jax
import jax.numpy as jnp
import numpy as np
from jax import lax
from jax.experimental import pallas as pl
from jax.experimental.pallas import tpu as pltpu
from jax.experimental.pallas import tpu_sc as plsc

N1, N2, NQ, NL = 2000, 2000, 4, 3
NP_, NN_, NZ_ = 100000, 100000, 800000
LOG2PI = float(np.log(2.0 * np.pi))

NC, NS = 2, 16
NW = NC * NS
SZ = 25008
SPN = 3136
BZ = SZ // 16
BPN = SPN // 16

_LOGC = (-7.788711203771115e-10, 0.9999999096875468, -0.49999956288490294,
         0.3333465033023849, -0.2500434801612538, 0.19950129931010036,
         -0.16511500339293791, 0.14932157138841426, -0.14707770124455374,
         0.09274272935687101)
_LN2 = 0.6931471805599453
_OFF = 0x3F330000


def _plog(g):
    bits = lax.bitcast_convert_type(g, jnp.int32)
    tmp = bits - jnp.int32(_OFF)
    k = tmp >> 23
    z = lax.bitcast_convert_type(bits - (tmp & jnp.int32(-8388608)), jnp.float32)
    u = z - 1.0
    acc = jnp.full(g.shape, _LOGC[-1], jnp.float32)
    for i in range(len(_LOGC) - 2, -1, -1):
        acc = acc * u + jnp.float32(_LOGC[i])
    return k.astype(jnp.float32) * jnp.float32(_LN2) + acc


def _tc_prep(g8_ref, t1_ref, t2_ref, sc_ref, tab_ref, dense_ref):
    g8 = g8_ref[...]
    ga, gra, gb, grb = g8[0:1], g8[1:2], g8[2:3], g8[3:4]
    gp, grp, gq, grq = g8[4:5], g8[5:6], g8[6:7], g8[7:8]
    rho_a, rho_b = jnp.exp(gra), jnp.exp(grb)
    rho_p, rho_q = jnp.exp(grp), jnp.exp(grq)

    zrow = jnp.zeros((1, N1), jnp.float32)
    t1 = jnp.concatenate([t1_ref[...], zrow], axis=0)
    mx1 = jnp.max(t1, axis=0, keepdims=True)
    e1 = jnp.exp(t1 - mx1)
    z1 = jnp.sum(e1, axis=0, keepdims=True)
    tau1 = e1 / z1
    S1 = jnp.sum(tau1 * (t1 - mx1 - jnp.log(z1)))

    t2 = jnp.concatenate([t2_ref[...], zrow], axis=0)
    mx2 = jnp.max(t2, axis=0, keepdims=True)
    e2 = jnp.exp(t2 - mx2)
    z2 = jnp.sum(e2, axis=0, keepdims=True)
    tau2 = e2 / z2
    S2 = jnp.sum(tau2 * (t2 - mx2 - jnp.log(z2)))

    entropy = (0.5 * (2 * N1 + 2 * N2) * (LOG2PI + 1.0)
               + 0.5 * (jnp.sum(gra) + jnp.sum(grb) + jnp.sum(grp) + jnp.sum(grq))
               - S1 - S2)
    ell_a = -N1 / 2 * (LOG2PI + sc_ref[1]) - sc_ref[5] * jnp.sum(rho_a + ga * ga)
    ell_b = -N1 / 2 * (LOG2PI + sc_ref[2]) - sc_ref[6] * jnp.sum(rho_b + gb * gb)
    ell_p = -N2 / 2 * (LOG2PI + sc_ref[3]) - sc_ref[7] * jnp.sum(rho_p + gp * gp)
    ell_q = -N2 / 2 * (LOG2PI + sc_ref[4]) - sc_ref[8] * jnp.sum(rho_q + gq * gq)
    ell_y1 = sum(jnp.sum(tau1[q:q + 1, :]) * sc_ref[33 + q] for q in range(NQ))
    ell_y2 = sum(jnp.sum(tau2[l:l + 1, :]) * sc_ref[37 + l] for l in range(NL))

    Ap = [sum(tau1[q:q + 1, :] * sc_ref[9 + q * NL + l] for q in range(NQ))
          for l in range(NL)]
    Am = [sum(tau1[q:q + 1, :] * sc_ref[21 + q * NL + l] for q in range(NQ))
          for l in range(NL)]

    tab_ref[...] = jnp.concatenate(
        [ga, gb, rho_a, rho_b, tau1, gp, gq, rho_p, rho_q, tau2] + Ap + Am,
        axis=0)
    dense_ref[0] = entropy + ell_a + ell_b + ell_p + ell_q + ell_y1 + ell_y2


def _sc_body(tab_h, c16_h, ip_h, jp_h, in_h, jn_h, iz_h, jz_h, out_h,
             tab_v, c_v, iz_v, jz_v, ip_v, jp_v, in_v, jn_v, acc_v, dsem):
    wid = lax.axis_index("s") * NC + lax.axis_index("c")
    copies = [
        pltpu.async_copy(tab_h, tab_v, dsem),
        pltpu.async_copy(c16_h, c_v, dsem),
        pltpu.async_copy(iz_h.at[pl.ds(wid * SZ, SZ)], iz_v, dsem),
        pltpu.async_copy(jz_h.at[pl.ds(wid * SZ, SZ)], jz_v, dsem),
        pltpu.async_copy(ip_h.at[pl.ds(wid * SPN, SPN)], ip_v, dsem),
        pltpu.async_copy(jp_h.at[pl.ds(wid * SPN, SPN)], jp_v, dsem),
        pltpu.async_copy(in_h.at[pl.ds(wid * SPN, SPN)], in_v, dsem),
        pltpu.async_copy(jn_h.at[pl.ds(wid * SPN, SPN)], jn_v, dsem),
    ]
    for cp in copies:
        cp.wait()

    def full(v):
        return jnp.full((16,), v, jnp.int32)

    def gath(row, ii):
        return plsc.load_gather(tab_v, [full(row), ii])


    mu_v = c_v[pl.ds(0, 16)]
    pi_vs = [c_v[pl.ds((1 + c) * 16, 16)] for c in range(NQ * NL)]
    iota = lax.iota(jnp.int32, 16)

    validz = NZ_ - wid * SZ

    def zbody(k, acc):
        ii = iz_v[pl.ds(k * 16, 16)]
        jj = jz_v[pl.ds(k * 16, 16)]
        na, nb = gath(0, ii), gath(1, ii)
        ra, rb = gath(2, ii), gath(3, ii)
        t1 = [gath(4 + q, ii) for q in range(NQ)]
        np_, nq_ = gath(8, jj), gath(9, jj)
        rp, rq = gath(10, jj), gath(11, jj)
        t2 = [gath(12 + l, jj) for l in range(NL)]
        x = na + np_
        y = nb + nq_
        svx = ra + rp
        svy = rb + rq
        s1 = 1.0 / (1.0 + jnp.exp(-(mu_v + x + y)))
        s2 = 1.0 / (1.0 + jnp.exp(-(mu_v + x - y)))
        u1 = s1 * (1.0 - s1)
        w1 = u1 * (1.0 - 2.0 * s1)
        u2 = s2 * (1.0 - s2)
        w2 = u2 * (1.0 - 2.0 * s2)
        A = 1.0 - s2
        d = s1 - s2
        nw2 = -w2
        ndw = w2 - w1
        nu2 = -u2
        ndu = u2 - u1
        nsu = -(u1 + u2)
        svs = 0.5 * (svx + svy)
        hx = 0.5 * svx
        hy = 0.5 * svy
        big = jnp.abs(d) >= 1e-5
        invd = 1.0 / jnp.where(big, d, 1.0)
        ndw_e = jnp.where(big, ndw, 0.0)
        ndu_e = jnp.where(big, ndu, 0.0)
        nsu_e = jnp.where(big, nsu, 0.0)
        Ad = A * invd
        C1 = -(ndw_e * invd)
        C2 = nw2 + ndw_e * Ad
        E1 = -(ndu_e * invd)
        E2 = nu2 + ndu_e * Ad
        F1 = -(nsu_e * invd)
        F2 = u2 + nsu_e * Ad
        hx2 = hx + hx
        hy2 = hy + hy
        P0 = svs * C1 - hx * (E1 * E1) - hy * (F1 * F1)
        P1 = svs * C2 - hx2 * (E1 * E2) - hy2 * (F1 * F2)
        P2 = -(hx * (E2 * E2) + hy * (F2 * F2))
        blk = jnp.zeros((16,), jnp.float32)
        for c in range(NQ * NL):
            piv = pi_vs[c]
            g = A - piv * d
            lg = _plog(g)
            r = 1.0 / g
            blk = blk + (t1[c // NL] * t2[c % NL]) * ((lg + P0) + (P1 + P2 * r) * r)
        return acc + jnp.where(iota < validz - k * 16, blk, 0.0)

    acc = plsc.parallel_loop(0, BZ, unroll=6,
                             carry=jnp.zeros((16,), jnp.float32))(
        lambda k, a: zbody(k, a))

    def make_pn(iv, jv, valid, arow, neg):
        def body(k, acc):
            ii = iv[pl.ds(k * 16, 16)]
            jj = jv[pl.ds(k * 16, 16)]
            na, nb = gath(0, ii), gath(1, ii)
            ra, rb = gath(2, ii), gath(3, ii)
            np_, nq_ = gath(8, jj), gath(9, jj)
            rp, rq = gath(10, jj), gath(11, jj)
            t2 = [gath(12 + l, jj) for l in range(NL)]
            A = [gath(arow + l, ii) for l in range(NL)]
            if neg:
                z = mu_v + na + np_ - nb - nq_
            else:
                z = mu_v + na + np_ + nb + nq_
            ez = jnp.exp(-jnp.abs(z))
            inv = 1.0 / (1.0 + ez)
            s = jnp.where(z >= 0, inv, ez * inv)
            lsg = jnp.minimum(z, 0.0) - _plog(1.0 + ez)
            der2 = -s * (1.0 - s)
            sv = ra + rb + rp + rq
            dotA = A[0] * t2[0] + A[1] * t2[1] + A[2] * t2[2]
            val = dotA + lsg + 0.5 * der2 * sv
            return acc + jnp.where(iota < valid - k * 16, val, 0.0)
        return body

    acc = plsc.parallel_loop(0, BPN, unroll=4, carry=acc)(
        make_pn(ip_v, jp_v, NP_ - wid * SPN, 15, False))
    acc = plsc.parallel_loop(0, BPN, unroll=4, carry=acc)(
        make_pn(in_v, jn_v, NN_ - wid * SPN, 18, True))

    acc_v[...] = acc
    pltpu.sync_copy(acc_v, out_h.at[pl.ds(wid * 16, 16)])


def _pad_idx(a, total):
    a = a.astype(jnp.int32)
    return jnp.pad(a, (0, total - a.shape[0]))


@jax.jit
def kernel(variationnal_params, model_params, i_p, j_p, i_n, j_n, i_z, j_z):
    vp = variationnal_params.astype(jnp.float32)
    mp = model_params.astype(jnp.float32)

    g8 = jnp.stack([vp[0:N1], vp[N1:2 * N1], vp[2 * N1:3 * N1], vp[3 * N1:4 * N1],
                    vp[4 * N1:4 * N1 + N2], vp[4 * N1 + N2:4 * N1 + 2 * N2],
                    vp[4 * N1 + 2 * N2:4 * N1 + 3 * N2],
                    vp[4 * N1 + 3 * N2:4 * N1 + 4 * N2]])
    o = 4 * N1 + 4 * N2
    t1T = vp[o:o + N1 * (NQ - 1)].reshape(N1, NQ - 1).T
    o += N1 * (NQ - 1)
    t2T = vp[o:o + N2 * (NL - 1)].reshape(N2, NL - 1).T

    mu = mp[0]
    inv2 = 0.5 * jnp.exp(-mp[1:5])
    la1 = jax.nn.log_softmax(jnp.pad(mp[5:5 + NQ - 1], (0, 1)))
    la2 = jax.nn.log_softmax(jnp.pad(mp[5 + NQ - 1:5 + NQ + NL - 2], (0, 1)))
    pi = jax.nn.sigmoid(mp[10:10 + NQ * NL])
    sc = jnp.concatenate([mp[0:5], inv2, jnp.log(pi), jnp.log1p(-pi), la1, la2])
    c16 = jnp.repeat(jnp.concatenate([mu[None], pi]), 16)

    tab, dense = pl.pallas_call(
        _tc_prep,
        in_specs=[pl.BlockSpec(memory_space=pltpu.VMEM),
                  pl.BlockSpec(memory_space=pltpu.VMEM),
                  pl.BlockSpec(memory_space=pltpu.VMEM),
                  pl.BlockSpec(memory_space=pltpu.SMEM)],
        out_specs=[pl.BlockSpec(memory_space=pltpu.VMEM),
                   pl.BlockSpec(memory_space=pltpu.SMEM)],
        out_shape=[jax.ShapeDtypeStruct((21, N1), jnp.float32),
                   jax.ShapeDtypeStruct((1,), jnp.float32)],
    )(g8, t1T, t2T, sc)

    sc_fn = pl.kernel(
        _sc_body,
        out_type=jax.ShapeDtypeStruct((NW * 16,), jnp.float32),
        mesh=plsc.VectorSubcoreMesh(core_axis_name="c", subcore_axis_name="s",
                                    num_cores=NC, num_subcores=NS),
        compiler_params=pltpu.CompilerParams(use_tc_tiling_on_sc=False, needs_layout_passes=False),
        scratch_types=[
            pltpu.VMEM((21, N1), jnp.float32),
            pltpu.VMEM((13 * 16,), jnp.float32),
            pltpu.VMEM((SZ,), jnp.int32),
            pltpu.VMEM((SZ,), jnp.int32),
            pltpu.VMEM((SPN,), jnp.int32),
            pltpu.VMEM((SPN,), jnp.int32),
            pltpu.VMEM((SPN,), jnp.int32),
            pltpu.VMEM((SPN,), jnp.int32),
            pltpu.VMEM((16,), jnp.float32),
            pltpu.SemaphoreType.DMA,
        ],
    )
    partials = sc_fn(tab, c16,
                     _pad_idx(i_p, NW * SPN), _pad_idx(j_p, NW * SPN),
                     _pad_idx(i_n, NW * SPN), _pad_idx(j_n, NW * SPN),
                     _pad_idx(i_z, NW * SZ), _pad_idx(j_z, NW * SZ))

    exp_x = jnp.sum(partials)
    exp_x = jnp.where(exp_x < 0, exp_x, jnp.inf)
    return -(dense + exp_x)
